# baseline (speedup 1.0000x reference)
import functools

import jax
import jax.numpy as jnp
import numpy as np
from jax import lax
from jax.experimental import pallas as pl
from jax.experimental.pallas import tpu as pltpu
from jax.experimental.pallas import tpu_sc as plsc
from jax._src import config as _jcfg

N = 10000
E = 160000
H = 256
RVD = 10
SH = H - RVD
NLAYER = 4
ATOM_VOCAB = 128
BOND_VOCAB = 8
NAF = 9
NBF = 3
BN_EPS = 1e-5

NC = 2
NS = 16
LANES = 16

Np = 10240
Ep = 163840
EB = 128
NPT_E = Ep // NS
NBLK_E = NPT_E // EB
NPT_N = Np // NS
NBLK_N = NPT_N // EB
EG = 64
SBE = 8
NSBE = NPT_E // (EG * SBE)
NZC = NPT_N // EG
CT_ROWS = BOND_VOCAB ** 3
HHALF = H // 2
VPR = HHALF // LANES


I0 = np.int32(0)


def _relu(x):
    return jnp.maximum(x, 0.0)


def _atom_body(at_hbm, nfidx_hbm, rxp_hbm, out_hbm,
               idx_v, gbuf, abuf, rep_v, sem):
    cid = lax.axis_index("c")
    sid = lax.axis_index("s")
    row0 = sid * np.int32(NPT_N)
    tbl_off = cid * np.int32(NAF * ATOM_VOCAB)

    @pl.loop(I0, np.int32(NBLK_N))
    def block(blk):
        base = row0 + blk * np.int32(EB)

        for i in range(NAF):
            pltpu.sync_copy(nfidx_hbm.at[pl.ds(np.int32(i * Np) + base, EB)], idx_v)

            @pl.loop(I0, np.int32(EB // LANES))
            def adj(v):
                sl = pl.ds(v * np.int32(LANES), LANES)
                idx_v[sl] = idx_v[sl] + tbl_off
            dstbuf = abuf if i == 0 else gbuf
            pltpu.async_copy(at_hbm.at[idx_v], dstbuf, sem).wait()
            if i > 0:
                @pl.loop(I0, np.int32(EB))
                def acc(r):
                    for v in range(VPR):
                        sl = pl.ds(v * LANES, LANES)
                        abuf[r, sl] = abuf[r, sl] + gbuf[r, sl]

        pltpu.sync_copy(rxp_hbm.at[pl.ds(base * np.int32(LANES), EB * LANES)], rep_v)

        @pl.loop(I0, np.int32(EB))
        def radd(r):
            sl = pl.ds((VPR - 1) * LANES, LANES)
            rv = jnp.where(cid == 1, rep_v[pl.ds(r * np.int32(LANES), LANES)], 0.0)
            abuf[r, sl] = abuf[r, sl] + rv

        pltpu.sync_copy(abuf, out_hbm.at[pl.ds(cid * np.int32(Np) + base, EB)])


def _atom_encode(at_cat, nfidx, rxp):
    mesh = plsc.VectorSubcoreMesh(core_axis_name="c", subcore_axis_name="s")
    fn = pl.kernel(
        _atom_body,
        out_type=jax.ShapeDtypeStruct((2 * Np, HHALF), jnp.float32),
        mesh=mesh,
        scratch_types=[
            pltpu.VMEM((EB,), jnp.int32),
            pltpu.VMEM((EB, HHALF), jnp.float32),
            pltpu.VMEM((EB, HHALF), jnp.float32),
            pltpu.VMEM((EB * LANES,), jnp.float32),
            pltpu.SemaphoreType.DMA,
        ],
    )
    return fn(at_cat, nfidx, rxp)


def _msg_body(h_hbm, ct_hbm, src_hbm, dst_hbm, code_hbm, rep_hbm, out_hbm,
              src_v, dst_v, code_v, rep_v, hbuf0, hbuf1, ebuf0, ebuf1,
              agg_sh, semh0, semh1, seme0, seme1, semsc0, semsc1):
    cid = lax.axis_index("c")
    sid = lax.axis_index("s")
    h_off = cid * np.int32(Np)
    ct_off = cid * np.int32(CT_ROWS)

    @pl.loop(I0, np.int32(EG))
    def zrow(r):
        for v in range(VPR):
            hbuf0[r, pl.ds(v * LANES, LANES)] = jnp.zeros((LANES,), jnp.float32)

    @pl.loop(I0, np.int32(NZC))
    def zchunk(k):
        pltpu.sync_copy(hbuf0, agg_sh.at[pl.ds(sid * np.int32(NPT_N) + k * np.int32(EG), EG)])
    plsc.subcore_barrier()

    hbufs = (hbuf0, hbuf1)
    ebufs = (ebuf0, ebuf1)
    semh = (semh0, semh1)
    seme = (seme0, seme1)
    semsc = (semsc0, semsc1)

    @pl.loop(I0, np.int32(NSBE))
    def superblk(sb):
        row0 = sid * np.int32(NPT_E // EG) + sb * np.int32(SBE)
        pltpu.sync_copy(src_hbm.at[pl.ds(row0, SBE)], src_v)
        pltpu.sync_copy(dst_hbm.at[pl.ds(row0, SBE)], dst_v)
        pltpu.sync_copy(code_hbm.at[pl.ds(row0, SBE)], code_v)
        pltpu.sync_copy(rep_hbm.at[pl.ds(row0, SBE)], rep_v)

        for j in range(SBE):
            @pl.loop(I0, np.int32(EG // LANES))
            def adj(v):
                sl = pl.ds(v * np.int32(LANES), LANES)
                src_v[j, sl] = src_v[j, sl] + h_off
                code_v[j, sl] = code_v[j, sl] + ct_off

        def start(j):
            b = j % 2
            cph = pltpu.async_copy(h_hbm.at[src_v.at[j]], hbufs[b], semh[b])
            cpe = pltpu.async_copy(ct_hbm.at[code_v.at[j]], ebufs[b], seme[b])
            return cph, cpe

        scat = [None] * SBE
        pending = start(0)
        for j in range(SBE):
            b = j % 2
            nxt = start(j + 1) if j + 1 < SBE else None
            pending[0].wait()
            pending[1].wait()
            if j >= 2:
                scat[j - 2].wait()
            hb, eb = hbufs[b], ebufs[b]

            @pl.loop(I0, np.int32(EG))
            def rows(r):
                for v in range(VPR):
                    sl = pl.ds(v * LANES, LANES)
                    x = hb[r, sl] + eb[r, sl]
                    if v == VPR - 1:
                        x = x + jnp.where(cid == 1,
                                          rep_v[j, pl.ds(r * np.int32(LANES), LANES)],
                                          0.0)
                    eb[r, sl] = _relu(x)

            scat[j] = pltpu.async_copy(eb, agg_sh.at[dst_v.at[j]], semsc[b],
                                       add=True)
            pending = nxt
        scat[SBE - 2].wait()
        scat[SBE - 1].wait()
    plsc.subcore_barrier()

    @pl.loop(I0, np.int32(NZC))
    def wchunk(k):
        rb = sid * np.int32(NPT_N) + k * np.int32(EG)
        pltpu.sync_copy(agg_sh.at[pl.ds(rb, EG)], hbuf0)
        pltpu.sync_copy(hbuf0, out_hbm.at[pl.ds(cid * np.int32(Np) + rb, EG)])


def _message_agg(h_cat, ct_cat, src, dst, code, rep):
    mesh = plsc.VectorSubcoreMesh(core_axis_name="c", subcore_axis_name="s")
    fn = pl.kernel(
        _msg_body,
        out_type=jax.ShapeDtypeStruct((2 * Np, HHALF), jnp.float32),
        mesh=mesh,
        scratch_types=[
            pltpu.VMEM((SBE, EG), jnp.int32),
            pltpu.VMEM((SBE, EG), jnp.int32),
            pltpu.VMEM((SBE, EG), jnp.int32),
            pltpu.VMEM((SBE, EG * LANES), jnp.float32),
            pltpu.VMEM((EG, HHALF), jnp.float32),
            pltpu.VMEM((EG, HHALF), jnp.float32),
            pltpu.VMEM((EG, HHALF), jnp.float32),
            pltpu.VMEM((EG, HHALF), jnp.float32),
            pltpu.VMEM_SHARED((Np, HHALF), jnp.float32),
            pltpu.SemaphoreType.DMA,
            pltpu.SemaphoreType.DMA,
            pltpu.SemaphoreType.DMA,
            pltpu.SemaphoreType.DMA,
            pltpu.SemaphoreType.DMA,
            pltpu.SemaphoreType.DMA,
        ],
    )
    return fn(h_cat, ct_cat, src, dst, code, rep)


RB = 2048
NG = Np // RB

_DOT = functools.partial(jnp.dot, preferred_element_type=jnp.float32,
                         precision=lax.Precision.HIGHEST)


def _acc_stats(g, x, acc_ref, stats_ref):
    rows = lax.broadcasted_iota(jnp.int32, (RB, 1), 0) + g * RB
    xm = jnp.where(rows < N, x, 0.0)
    ps = jnp.sum(xm, axis=0, keepdims=True)
    pq = jnp.sum(xm * xm, axis=0, keepdims=True)
    blk = jnp.concatenate([ps, pq], axis=0)

    @pl.when(g == 0)
    def _():
        acc_ref[...] = blk

    @pl.when(g > 0)
    def _():
        acc_ref[...] = acc_ref[...] + blk

    @pl.when(g == NG - 1)
    def _():
        stats_ref[...] = acc_ref[...]


def _norm_from_stats(stats_ref, g_ref, b_ref):
    mu = stats_ref[0:1, :] / N
    var = stats_ref[1:2, :] / N - mu * mu
    c1 = lax.rsqrt(var + BN_EPS) * g_ref[...]
    c0 = b_ref[...] - mu * c1
    return c1, c0


def _mlp1_body(scale_ref, ht, hb, at, ab, w1_ref, b1_ref, a_out, stats_ref,
               acc_ref):
    g = pl.program_id(0)
    s = scale_ref[0, 0]
    zt = s * ht[...] + at[...]
    zb = s * hb[...] + ab[...]
    a = _DOT(zt, w1_ref[0:HHALF, :]) + _DOT(zb, w1_ref[HHALF:H, :])
    a = a + b1_ref[...]
    a_out[...] = a
    _acc_stats(g, a, acc_ref, stats_ref)


def _mlp2_body(stats1_ref, g1_ref, bb1_ref, a_ref, w2_ref, b2_ref, b_out,
               stats_ref, acc_ref):
    g = pl.program_id(0)
    c1, c0 = _norm_from_stats(stats1_ref, g1_ref, bb1_ref)
    t = _relu(a_ref[...] * c1 + c0)
    b = _DOT(t, w2_ref[...]) + b2_ref[...]
    b_out[...] = b
    _acc_stats(g, b, acc_ref, stats_ref)


def _mlp3_body(last, stats2_ref, g2_ref, bb2_ref, b_ref, o1_ref, o2_ref=None):
    c1, c0 = _norm_from_stats(stats2_ref, g2_ref, bb2_ref)
    y = b_ref[...] * c1 + c0
    if last:
        o1_ref[...] = y
    else:
        y = _relu(y)
        o1_ref[...] = y[:, 0:HHALF]
        o2_ref[...] = y[:, HHALF:H]


def _vspec(shape):
    return pl.BlockSpec(shape, lambda g: (0, 0))


def _mlp(h_cat, agg_cat, scale, w1, b1, g1, bb1, w2, b2, g2, bb2, last):
    f32 = jnp.float32
    blk = pl.BlockSpec((RB, H), lambda g: (g, 0))
    top = pl.BlockSpec((RB, HHALF), lambda g: (g, 0))
    bot = pl.BlockSpec((RB, HHALF), lambda g: (g + NG, 0))

    a, stats1 = pl.pallas_call(
        _mlp1_body,
        grid=(NG,),
        in_specs=[_vspec((1, 1)), top, bot, top, bot, _vspec((H, H)),
                  _vspec((1, H))],
        out_specs=[blk, _vspec((2, H))],
        out_shape=[jax.ShapeDtypeStruct((Np, H), f32),
                   jax.ShapeDtypeStruct((2, H), f32)],
        scratch_shapes=[pltpu.VMEM((2, H), f32)],
    )(scale, h_cat, h_cat, agg_cat, agg_cat, w1, b1)

    b, stats2 = pl.pallas_call(
        _mlp2_body,
        grid=(NG,),
        in_specs=[_vspec((2, H)), _vspec((1, H)), _vspec((1, H)), blk,
                  _vspec((H, H)), _vspec((1, H))],
        out_specs=[blk, _vspec((2, H))],
        out_shape=[jax.ShapeDtypeStruct((Np, H), f32),
                   jax.ShapeDtypeStruct((2, H), f32)],
        scratch_shapes=[pltpu.VMEM((2, H), f32)],
    )(stats1, g1, bb1, a, w2, b2)

    if last:
        y = pl.pallas_call(
            functools.partial(_mlp3_body, True),
            grid=(NG,),
            in_specs=[_vspec((2, H)), _vspec((1, H)), _vspec((1, H)), blk],
            out_specs=[blk],
            out_shape=[jax.ShapeDtypeStruct((Np, H), f32)],
        )(stats2, g2, bb2, b)[0]
        return y[0:N, :]
    o1, o2 = pl.pallas_call(
        functools.partial(_mlp3_body, False),
        grid=(NG,),
        in_specs=[_vspec((2, H)), _vspec((1, H)), _vspec((1, H)), blk],
        out_specs=[top, top],
        out_shape=[jax.ShapeDtypeStruct((Np, HHALF), f32),
                   jax.ShapeDtypeStruct((Np, HHALF), f32)],
    )(stats2, g2, bb2, b)
    return jnp.concatenate([o1, o2], axis=0)


def kernel(rand_x, rand_edge, edge_index, node_feat, edge_attr, atom_tables,
           bond_tables, eps, W1, b1, bn1_g, bn1_b, W2, b2, out_bn_g,
           out_bn_b):
    with _jcfg.enable_x64(False):
        out = _kernel_32(rand_x, rand_edge, edge_index, node_feat,
                         edge_attr, atom_tables, bond_tables, eps, W1, b1,
                         bn1_g, bn1_b, W2, b2, out_bn_g, out_bn_b)
    return out.astype(jnp.float64)


def _kernel_32(rand_x, rand_edge, edge_index, node_feat, edge_attr,
               atom_tables, bond_tables, eps, W1, b1, bn1_g, bn1_b, W2, b2,
               out_bn_g, out_bn_b):
    f32 = jnp.float32
    i32 = jnp.int32

    src = edge_index[0].astype(i32)
    dst = edge_index[1].astype(i32)
    ea = edge_attr.astype(i32)
    code = ea[:, 0] * (BOND_VOCAB * BOND_VOCAB) + ea[:, 1] * BOND_VOCAB + ea[:, 2]

    src_p = jnp.concatenate([src, jnp.zeros((Ep - E,), i32)])
    dst_p = jnp.concatenate([dst, jnp.full((Ep - E,), Np - 1, i32)])
    code_p = jnp.concatenate([code, jnp.zeros((Ep - E,), i32)])

    rep = jnp.concatenate(
        [jnp.zeros((E, LANES - RVD), f32), rand_edge.astype(f32)], axis=1)
    rep = jnp.concatenate([rep, jnp.zeros((Ep - E, LANES), f32)], axis=0)
    rep = rep.reshape(Ep // EG, EG * LANES)
    src_p = src_p.reshape(Ep // EG, EG)
    dst_p = dst_p.reshape(Ep // EG, EG)
    code_p = code_p.reshape(Ep // EG, EG)

    rxp = jnp.concatenate(
        [jnp.zeros((N, LANES - RVD), f32), rand_x.astype(f32)], axis=1)
    rxp = jnp.concatenate([rxp, jnp.zeros((Np - N, LANES), f32)], axis=0)
    rxp = rxp.reshape(-1)

    nf = node_feat.astype(i32)
    nf = jnp.concatenate([nf, jnp.zeros((Np - N, NAF), i32)], axis=0)
    nfidx = (nf.T + ATOM_VOCAB * jnp.arange(NAF, dtype=i32)[:, None]).reshape(-1)

    atp = jnp.concatenate(
        [atom_tables.astype(f32),
         jnp.zeros((NAF, ATOM_VOCAB, H - SH), f32)], axis=2)
    atp = atp.reshape(NAF * ATOM_VOCAB, H)
    at_cat = jnp.concatenate([atp[:, :HHALF], atp[:, HHALF:]], axis=0)

    bt = bond_tables.astype(f32)
    ct = (bt[:, 0, :, None, None, :] + bt[:, 1, None, :, None, :]
          + bt[:, 2, None, None, :, :])
    ct = ct.reshape(NLAYER, BOND_VOCAB ** 3, SH)
    ct = jnp.concatenate(
        [ct, jnp.zeros((NLAYER, BOND_VOCAB ** 3, H - SH), f32)], axis=2)
    ct_cat = jnp.concatenate([ct[:, :, :HHALF], ct[:, :, HHALF:]], axis=1)

    h_cat = _atom_encode(at_cat, nfidx, rxp)
    for l in range(NLAYER):
        agg_cat = _message_agg(h_cat, ct_cat[l], src_p, dst_p, code_p, rep)
        scale = (1.0 + eps[l]).astype(f32).reshape(1, 1)
        h_cat = _mlp(h_cat, agg_cat, scale,
                     W1[l].astype(f32), b1[l].reshape(1, H).astype(f32),
                     bn1_g[l].reshape(1, H).astype(f32),
                     bn1_b[l].reshape(1, H).astype(f32),
                     W2[l].astype(f32), b2[l].reshape(1, H).astype(f32),
                     out_bn_g[l].reshape(1, H).astype(f32),
                     out_bn_b[l].reshape(1, H).astype(f32),
                     last=(l == NLAYER - 1))
    return h_cat


# speedup vs baseline: 1.3030x; 1.3030x over previous
import functools

import jax
import jax.numpy as jnp
import numpy as np
from jax import lax
from jax.experimental import pallas as pl
from jax.experimental.pallas import tpu as pltpu
from jax.experimental.pallas import tpu_sc as plsc
from jax._src import config as _jcfg

N = 10000
E = 160000
H = 256
RVD = 10
SH = H - RVD
NLAYER = 4
ATOM_VOCAB = 128
BOND_VOCAB = 8
NAF = 9
NBF = 3
BN_EPS = 1e-5

NC = 2
NS = 16
LANES = 16

Np = 10240
Ep = 163840
EB = 128
NPT_E = Ep // NS
NBLK_E = NPT_E // EB
NPT_N = Np // NS
NBLK_N = NPT_N // EB
EG = 64
SBE = 8
NSBE = NPT_E // (EG * SBE)
NZC2 = NPT_N // EG
CT_ROWS = BOND_VOCAB ** 3
HHALF = H // 2
VPR = HHALF // LANES


I0 = np.int32(0)


def _relu(x):
    return jnp.maximum(x, 0.0)


def _atom_body(at_hbm, nfidx_hbm, rxp_hbm, out_hbm,
               idx_v, gbufa, gbufb, abuf, rep_v, sema, semb):
    cid = lax.axis_index("c")
    sid = lax.axis_index("s")
    row0 = sid * np.int32(NPT_N)
    tbl_off = cid * np.int32(NAF * ATOM_VOCAB)
    gbufs = (gbufa, gbufb)
    sems = (sema, semb)

    @pl.loop(I0, np.int32(NBLK_N))
    def block(blk):
        base = row0 + blk * np.int32(EB)

        brow = (base // np.int32(EB)) * np.int32(16)
        pltpu.sync_copy(nfidx_hbm.at[pl.ds(brow, 16)], idx_v)

        for i in range(NAF):
            @pl.loop(I0, np.int32(EB // LANES))
            def adj(v):
                sl = pl.ds(v * np.int32(LANES), LANES)
                idx_v[i, sl] = idx_v[i, sl] + tbl_off

        def start(i):
            dstbuf = abuf if i == 0 else gbufs[i % 2]
            return pltpu.async_copy(at_hbm.at[idx_v.at[i]], dstbuf,
                                    sems[i % 2])

        pending = start(0)
        for i in range(NAF):
            nxt = start(i + 1) if i + 1 < NAF else None
            pending.wait()
            if i > 0:
                g = gbufs[i % 2]

                @pl.loop(I0, np.int32(EB), unroll=2)
                def acc(r):
                    for v in range(VPR):
                        sl = pl.ds(v * LANES, LANES)
                        abuf[r, sl] = abuf[r, sl] + g[r, sl]
            pending = nxt

        pltpu.sync_copy(rxp_hbm.at[pl.ds(base * np.int32(LANES), EB * LANES)], rep_v)

        @pl.loop(I0, np.int32(EB))
        def radd(r):
            sl = pl.ds((VPR - 1) * LANES, LANES)
            rv = jnp.where(cid == 1, rep_v[pl.ds(r * np.int32(LANES), LANES)], 0.0)
            abuf[r, sl] = abuf[r, sl] + rv

        pltpu.sync_copy(abuf, out_hbm.at[pl.ds(cid * np.int32(Np) + base, EB)])


def _atom_encode(at_cat, nfidx, rxp):
    mesh = plsc.VectorSubcoreMesh(core_axis_name="c", subcore_axis_name="s")
    fn = pl.kernel(
        _atom_body,
        out_type=jax.ShapeDtypeStruct((2 * Np, HHALF), jnp.float32),
        mesh=mesh,
        scratch_types=[
            pltpu.VMEM((16, EB), jnp.int32),
            pltpu.VMEM((EB, HHALF), jnp.float32),
            pltpu.VMEM((EB, HHALF), jnp.float32),
            pltpu.VMEM((EB, HHALF), jnp.float32),
            pltpu.VMEM((EB * LANES,), jnp.float32),
            pltpu.SemaphoreType.DMA,
            pltpu.SemaphoreType.DMA,
        ],
    )
    return fn(at_cat, nfidx, rxp)


def _msg_body(big_hbm, idx_hbm, dst_hbm, rep_hbm, out_hbm,
              idx_v, dst_v, rep_v, gbuf0, gbuf1,
              agg_sh, semg0, semg1, semsc0, semsc1):
    cid = lax.axis_index("c")
    sid = lax.axis_index("s")
    h_off = cid * np.int32(Np)
    ct_off = cid * np.int32(CT_ROWS)

    @pl.loop(I0, np.int32(EG))
    def zrow(r):
        for v in range(VPR):
            gbuf0[r, pl.ds(v * LANES, LANES)] = jnp.zeros((LANES,), jnp.float32)

    @pl.loop(I0, np.int32(NZC2))
    def zchunk(k):
        pltpu.sync_copy(gbuf0.at[pl.ds(0, EG), pl.ds(0, HHALF)],
                        agg_sh.at[pl.ds(sid * np.int32(NPT_N) + k * np.int32(EG), EG)])
    plsc.subcore_barrier()

    gbufs = (gbuf0, gbuf1)
    semg = (semg0, semg1)
    semsc = (semsc0, semsc1)

    @pl.loop(I0, np.int32(NSBE))
    def superblk(sb):
        row0 = sid * np.int32(NPT_E // EG) + sb * np.int32(SBE)
        pltpu.sync_copy(idx_hbm.at[pl.ds(row0, SBE)], idx_v)
        pltpu.sync_copy(dst_hbm.at[pl.ds(row0, SBE)], dst_v)
        pltpu.sync_copy(rep_hbm.at[pl.ds(row0, SBE)], rep_v)

        for j in range(SBE):
            @pl.loop(I0, np.int32(EG // LANES))
            def adj(v):
                sl = pl.ds(v * np.int32(LANES), LANES)
                sl2 = pl.ds(np.int32(EG) + v * np.int32(LANES), LANES)
                idx_v[j, sl] = idx_v[j, sl] + h_off
                idx_v[j, sl2] = idx_v[j, sl2] + ct_off

        def start(j):
            b = j % 2
            return pltpu.async_copy(big_hbm.at[idx_v.at[j]], gbufs[b], semg[b])

        scat = [None] * SBE
        pending = start(0)
        for j in range(SBE):
            b = j % 2
            if j >= 1:
                scat[j - 1].wait()
            nxt = start(j + 1) if j + 1 < SBE else None
            pending.wait()
            g = gbufs[b]

            @pl.loop(I0, np.int32(EG), unroll=2)
            def rows(r):
                for v in range(VPR):
                    sl = pl.ds(v * LANES, LANES)
                    x = g[r, sl] + g[np.int32(EG) + r, sl]
                    if v == VPR - 1:
                        x = x + jnp.where(cid == 1,
                                          rep_v[j, pl.ds(r * np.int32(LANES), LANES)],
                                          0.0)
                    g[np.int32(EG) + r, sl] = _relu(x)

            scat[j] = pltpu.async_copy(g.at[pl.ds(EG, EG)],
                                       agg_sh.at[dst_v.at[j]], semsc[b],
                                       add=True)
            pending = nxt
        scat[SBE - 1].wait()
    plsc.subcore_barrier()

    @pl.loop(I0, np.int32(NZC2))
    def wchunk(k):
        rb = sid * np.int32(NPT_N) + k * np.int32(EG)
        pltpu.sync_copy(agg_sh.at[pl.ds(rb, EG)],
                        gbuf0.at[pl.ds(0, EG), pl.ds(0, HHALF)])
        pltpu.sync_copy(gbuf0.at[pl.ds(0, EG), pl.ds(0, HHALF)],
                        out_hbm.at[pl.ds(cid * np.int32(Np) + rb, EG)])


def _message_agg(big, idx, dst, rep):
    mesh = plsc.VectorSubcoreMesh(core_axis_name="c", subcore_axis_name="s")
    fn = pl.kernel(
        _msg_body,
        out_type=jax.ShapeDtypeStruct((2 * Np, HHALF), jnp.float32),
        mesh=mesh,
        scratch_types=[
            pltpu.VMEM((SBE, 2 * EG), jnp.int32),
            pltpu.VMEM((SBE, EG), jnp.int32),
            pltpu.VMEM((SBE, EG * LANES), jnp.float32),
            pltpu.VMEM((2 * EG, HHALF), jnp.float32),
            pltpu.VMEM((2 * EG, HHALF), jnp.float32),
            pltpu.VMEM_SHARED((Np, HHALF), jnp.float32),
            pltpu.SemaphoreType.DMA,
            pltpu.SemaphoreType.DMA,
            pltpu.SemaphoreType.DMA,
            pltpu.SemaphoreType.DMA,
        ],
    )
    return fn(big, idx, dst, rep)


RB = 2048
NG = Np // RB

_DOT = functools.partial(jnp.dot, preferred_element_type=jnp.float32,
                         precision=lax.Precision.HIGHEST)


def _acc_stats(g, x, acc_ref, stats_ref):
    rows = lax.broadcasted_iota(jnp.int32, (RB, 1), 0) + g * RB
    xm = jnp.where(rows < N, x, 0.0)
    ps = jnp.sum(xm, axis=0, keepdims=True)
    pq = jnp.sum(xm * xm, axis=0, keepdims=True)
    blk = jnp.concatenate([ps, pq], axis=0)

    @pl.when(g == 0)
    def _():
        acc_ref[...] = blk

    @pl.when(g > 0)
    def _():
        acc_ref[...] = acc_ref[...] + blk

    @pl.when(g == NG - 1)
    def _():
        stats_ref[...] = acc_ref[...]


def _norm_from_stats(stats_ref, g_ref, b_ref):
    mu = stats_ref[0:1, :] / N
    var = stats_ref[1:2, :] / N - mu * mu
    c1 = lax.rsqrt(var + BN_EPS) * g_ref[...]
    c0 = b_ref[...] - mu * c1
    return c1, c0


def _mlp1_body(scale_ref, ht, hb, at, ab, w1_ref, b1_ref, a_out, stats_ref,
               acc_ref):
    g = pl.program_id(0)
    s = scale_ref[0, 0]
    zt = s * ht[...] + at[...]
    zb = s * hb[...] + ab[...]
    a = _DOT(zt, w1_ref[0:HHALF, :]) + _DOT(zb, w1_ref[HHALF:H, :])
    a = a + b1_ref[...]
    a_out[...] = a
    _acc_stats(g, a, acc_ref, stats_ref)


def _mlp2_body(stats1_ref, g1_ref, bb1_ref, a_ref, w2_ref, b2_ref, b_out,
               stats_ref, acc_ref):
    g = pl.program_id(0)
    c1, c0 = _norm_from_stats(stats1_ref, g1_ref, bb1_ref)
    t = _relu(a_ref[...] * c1 + c0)
    b = _DOT(t, w2_ref[...]) + b2_ref[...]
    b_out[...] = b
    _acc_stats(g, b, acc_ref, stats_ref)


def _mlp3_body(last, stats2_ref, g2_ref, bb2_ref, b_ref, o1_ref, o2_ref=None):
    c1, c0 = _norm_from_stats(stats2_ref, g2_ref, bb2_ref)
    y = b_ref[...] * c1 + c0
    if last:
        o1_ref[...] = y
    else:
        y = _relu(y)
        o1_ref[...] = y[:, 0:HHALF]
        o2_ref[...] = y[:, HHALF:H]


def _vspec(shape):
    return pl.BlockSpec(shape, lambda g: (0, 0))


def _mlp(h_cat, agg_cat, scale, w1, b1, g1, bb1, w2, b2, g2, bb2, last):
    f32 = jnp.float32
    blk = pl.BlockSpec((RB, H), lambda g: (g, 0))
    top = pl.BlockSpec((RB, HHALF), lambda g: (g, 0))
    bot = pl.BlockSpec((RB, HHALF), lambda g: (g + NG, 0))

    a, stats1 = pl.pallas_call(
        _mlp1_body,
        grid=(NG,),
        in_specs=[_vspec((1, 1)), top, bot, top, bot, _vspec((H, H)),
                  _vspec((1, H))],
        out_specs=[blk, _vspec((2, H))],
        out_shape=[jax.ShapeDtypeStruct((Np, H), f32),
                   jax.ShapeDtypeStruct((2, H), f32)],
        scratch_shapes=[pltpu.VMEM((2, H), f32)],
    )(scale, h_cat, h_cat, agg_cat, agg_cat, w1, b1)

    b, stats2 = pl.pallas_call(
        _mlp2_body,
        grid=(NG,),
        in_specs=[_vspec((2, H)), _vspec((1, H)), _vspec((1, H)), blk,
                  _vspec((H, H)), _vspec((1, H))],
        out_specs=[blk, _vspec((2, H))],
        out_shape=[jax.ShapeDtypeStruct((Np, H), f32),
                   jax.ShapeDtypeStruct((2, H), f32)],
        scratch_shapes=[pltpu.VMEM((2, H), f32)],
    )(stats1, g1, bb1, a, w2, b2)

    if last:
        y = pl.pallas_call(
            functools.partial(_mlp3_body, True),
            grid=(NG,),
            in_specs=[_vspec((2, H)), _vspec((1, H)), _vspec((1, H)), blk],
            out_specs=[blk],
            out_shape=[jax.ShapeDtypeStruct((Np, H), f32)],
        )(stats2, g2, bb2, b)[0]
        return y[0:N, :]
    o1, o2 = pl.pallas_call(
        functools.partial(_mlp3_body, False),
        grid=(NG,),
        in_specs=[_vspec((2, H)), _vspec((1, H)), _vspec((1, H)), blk],
        out_specs=[top, top],
        out_shape=[jax.ShapeDtypeStruct((Np, HHALF), f32),
                   jax.ShapeDtypeStruct((Np, HHALF), f32)],
    )(stats2, g2, bb2, b)
    return jnp.concatenate([o1, o2], axis=0)


def kernel(rand_x, rand_edge, edge_index, node_feat, edge_attr, atom_tables,
           bond_tables, eps, W1, b1, bn1_g, bn1_b, W2, b2, out_bn_g,
           out_bn_b):
    with _jcfg.enable_x64(False):
        out = _kernel_32(rand_x, rand_edge, edge_index, node_feat,
                         edge_attr, atom_tables, bond_tables, eps, W1, b1,
                         bn1_g, bn1_b, W2, b2, out_bn_g, out_bn_b)
    return out.astype(jnp.float64)


def _kernel_32(rand_x, rand_edge, edge_index, node_feat, edge_attr,
               atom_tables, bond_tables, eps, W1, b1, bn1_g, bn1_b, W2, b2,
               out_bn_g, out_bn_b):
    f32 = jnp.float32
    i32 = jnp.int32

    src = edge_index[0].astype(i32)
    dst = edge_index[1].astype(i32)
    ea = edge_attr.astype(i32)
    code = ea[:, 0] * (BOND_VOCAB * BOND_VOCAB) + ea[:, 1] * BOND_VOCAB + ea[:, 2]

    src_p = jnp.concatenate([src, jnp.zeros((Ep - E,), i32)])
    dst_p = jnp.concatenate([dst, jnp.full((Ep - E,), Np - 1, i32)])
    code_p = jnp.concatenate([code, jnp.zeros((Ep - E,), i32)])

    rep = jnp.concatenate(
        [jnp.zeros((E, LANES - RVD), f32), rand_edge.astype(f32)], axis=1)
    rep = jnp.concatenate([rep, jnp.zeros((Ep - E, LANES), f32)], axis=0)
    rep = rep.reshape(Ep // EG, EG * LANES)
    idxc = jnp.concatenate(
        [src_p.reshape(Ep // EG, EG),
         (code_p + 2 * Np).reshape(Ep // EG, EG)], axis=1)
    dst_p = dst_p.reshape(Ep // EG, EG)

    rxp = jnp.concatenate(
        [jnp.zeros((N, LANES - RVD), f32), rand_x.astype(f32)], axis=1)
    rxp = jnp.concatenate([rxp, jnp.zeros((Np - N, LANES), f32)], axis=0)
    rxp = rxp.reshape(-1)

    nf = node_feat.astype(i32)
    nf = jnp.concatenate([nf, jnp.zeros((Np - N, NAF), i32)], axis=0)
    nfidx = nf + ATOM_VOCAB * jnp.arange(NAF, dtype=i32)[None, :]
    nfidx = nfidx.reshape(Np // EB, EB, NAF).transpose(0, 2, 1)
    nfidx = jnp.concatenate(
        [nfidx, jnp.zeros((Np // EB, 16 - NAF, EB), i32)], axis=1)
    nfidx = nfidx.reshape(Np // EB * 16, EB)

    atp = jnp.concatenate(
        [atom_tables.astype(f32),
         jnp.zeros((NAF, ATOM_VOCAB, H - SH), f32)], axis=2)
    atp = atp.reshape(NAF * ATOM_VOCAB, H)
    at_cat = jnp.concatenate([atp[:, :HHALF], atp[:, HHALF:]], axis=0)

    bt = bond_tables.astype(f32)
    ct = (bt[:, 0, :, None, None, :] + bt[:, 1, None, :, None, :]
          + bt[:, 2, None, None, :, :])
    ct = ct.reshape(NLAYER, BOND_VOCAB ** 3, SH)
    ct = jnp.concatenate(
        [ct, jnp.zeros((NLAYER, BOND_VOCAB ** 3, H - SH), f32)], axis=2)
    ct_cat = jnp.concatenate([ct[:, :, :HHALF], ct[:, :, HHALF:]], axis=1)

    h_cat = _atom_encode(at_cat, nfidx, rxp)
    for l in range(NLAYER):
        big = jnp.concatenate([h_cat, ct_cat[l]], axis=0)
        agg_cat = _message_agg(big, idxc, dst_p, rep)
        scale = (1.0 + eps[l]).astype(f32).reshape(1, 1)
        h_cat = _mlp(h_cat, agg_cat, scale,
                     W1[l].astype(f32), b1[l].reshape(1, H).astype(f32),
                     bn1_g[l].reshape(1, H).astype(f32),
                     bn1_b[l].reshape(1, H).astype(f32),
                     W2[l].astype(f32), b2[l].reshape(1, H).astype(f32),
                     out_bn_g[l].reshape(1, H).astype(f32),
                     out_bn_b[l].reshape(1, H).astype(f32),
                     last=(l == NLAYER - 1))
    return h_cat
